# Initial kernel scaffold; baseline (speedup 1.0000x reference)
import jax
import jax.numpy as jnp
from jax.experimental import pallas as pl


def kernel(x, edge_index, edge_weight, W_conv, b_conv, W1, b1, W2, b2, W3, b3, W4, b4, W5, b5):
    raise NotImplementedError("write your pallas kernel here")


# speedup vs baseline: 146.5415x; 146.5415x over previous
import functools

import jax
import jax.numpy as jnp
from jax import lax
from jax.experimental import pallas as pl
from jax.experimental.pallas import tpu as pltpu
from jax.experimental.pallas import tpu_sc as plsc

NC = 2
NS = 16
LANES = 128
R = 16


def _row_split(rows, w):
    nw = NC * NS
    blocks = rows // 8
    base = blocks // nw
    extra = blocks % nw
    start = 8 * (w * base + jnp.minimum(w, extra))
    nrows = 8 * (base + (w < extra).astype(jnp.int32))
    return start, nrows


def _deg_kernel_body(npad, span, rows, dst_hbm, w_hbm, out_hbm,
                     dbuf, wbuf, zbuf, deg_sh):
    c = lax.axis_index("c")
    s = lax.axis_index("s")
    w = c * NS + s

    def zb(i, _):
        zbuf[pl.ds(i * 16, 16)] = jnp.zeros((16,), jnp.float32)
        return 0
    lax.fori_loop(0, span // 16, zb, 0)
    pltpu.sync_copy(zbuf, deg_sh.at[pl.ds(s * span, span)])
    plsc.subcore_barrier()

    start, nrows = _row_split(rows, w)
    nfull = nrows // R
    rem = nrows - nfull * R

    def chunk(k, _):
        r0 = start + k * R
        pltpu.sync_copy(dst_hbm.at[pl.ds(r0, R)], dbuf)
        pltpu.sync_copy(w_hbm.at[pl.ds(r0, R)], wbuf)
        for i in range(R):
            pltpu.sync_copy(wbuf.at[i], deg_sh.at[dbuf.at[i]], add=True)
        return 0
    lax.fori_loop(0, nfull, chunk, 0)

    def tail(k, _):
        r0 = start + nfull * R + k * 8
        pltpu.sync_copy(dst_hbm.at[pl.ds(r0, 8)], dbuf.at[pl.ds(0, 8)])
        pltpu.sync_copy(w_hbm.at[pl.ds(r0, 8)], wbuf.at[pl.ds(0, 8)])
        for i in range(8):
            pltpu.sync_copy(wbuf.at[i], deg_sh.at[dbuf.at[i]], add=True)
        return 0
    lax.fori_loop(0, rem // 8, tail, 0)

    plsc.subcore_barrier()
    pltpu.sync_copy(deg_sh.at[pl.ds(s * span, span)],
                    out_hbm.at[pl.ds(c * npad + s * span, span)])


def _t_kernel_body(npad, span, rows, src_hbm, dst_hbm, w_hbm, u_hbm, out_hbm,
                   sbuf, dbuf, wbuf, pbuf, zbuf, u_v, t_sh):
    c = lax.axis_index("c")
    s = lax.axis_index("s")
    w = c * NS + s

    def zb(i, _):
        zbuf[pl.ds(i * 16, 16)] = jnp.zeros((16,), jnp.float32)
        return 0
    lax.fori_loop(0, span // 16, zb, 0)
    pltpu.sync_copy(zbuf, t_sh.at[pl.ds(s * span, span)])
    pltpu.sync_copy(u_hbm, u_v)
    plsc.subcore_barrier()

    start, nrows = _row_split(rows, w)
    nfull = nrows // R
    rem = nrows - nfull * R

    def gather_mul_row(i):
        for j in range(LANES // 16):
            sl = pl.ds(j * 16, 16)
            idx = sbuf[i, sl]
            g = plsc.load_gather(u_v, [idx])
            pbuf[i, sl] = g * wbuf[i, sl]

    def chunk(k, _):
        r0 = start + k * R
        pltpu.sync_copy(src_hbm.at[pl.ds(r0, R)], sbuf)
        pltpu.sync_copy(dst_hbm.at[pl.ds(r0, R)], dbuf)
        pltpu.sync_copy(w_hbm.at[pl.ds(r0, R)], wbuf)
        for i in range(R):
            gather_mul_row(i)
        for i in range(R):
            pltpu.sync_copy(pbuf.at[i], t_sh.at[dbuf.at[i]], add=True)
        return 0
    lax.fori_loop(0, nfull, chunk, 0)

    def tail(k, _):
        r0 = start + nfull * R + k * 8
        pltpu.sync_copy(src_hbm.at[pl.ds(r0, 8)], sbuf.at[pl.ds(0, 8)])
        pltpu.sync_copy(dst_hbm.at[pl.ds(r0, 8)], dbuf.at[pl.ds(0, 8)])
        pltpu.sync_copy(w_hbm.at[pl.ds(r0, 8)], wbuf.at[pl.ds(0, 8)])
        for i in range(8):
            gather_mul_row(i)
            pltpu.sync_copy(pbuf.at[i], t_sh.at[dbuf.at[i]], add=True)
        return 0
    lax.fori_loop(0, rem // 8, tail, 0)

    plsc.subcore_barrier()
    pltpu.sync_copy(t_sh.at[pl.ds(s * span, span)],
                    out_hbm.at[pl.ds(c * npad + s * span, span)])


def _mlp_kernel_body(zr, wc64r, bc64r, e64r,
                     w1r, b1r, w2r, b2r, w3r, b3r, w4r, b4r, w5r, b5r, outr):
    sb = zr[..., 0:1]
    xb = zr[..., 1:2]
    z = sb * wc64r[...] + xb * e64r[...] + bc64r[...]
    h = jnp.maximum(jnp.dot(z, w1r[...]) + b1r[...], 0.0)
    h = jnp.maximum(jnp.dot(h, w2r[...]) + b2r[...], 0.0)
    h = jnp.maximum(jnp.dot(h, w3r[...]) + b3r[...], 0.0)
    h = jnp.maximum(jnp.dot(h, w4r[...]) + b4r[...], 0.0)
    outr[...] = jnp.dot(h, w5r[...]) + b5r[...]


@functools.partial(jax.jit, static_argnames=("npad", "span", "rows"))
def _run_sc_deg(dst2d, w2d, *, npad, span, rows):
    mesh = plsc.VectorSubcoreMesh(core_axis_name="c", subcore_axis_name="s")
    body = functools.partial(_deg_kernel_body, npad, span, rows)
    return pl.kernel(
        body,
        out_type=jax.ShapeDtypeStruct((NC * npad,), jnp.float32),
        mesh=mesh,
        compiler_params=pltpu.CompilerParams(needs_layout_passes=False),
        scratch_types=[
            pltpu.VMEM((R, LANES), jnp.int32),
            pltpu.VMEM((R, LANES), jnp.float32),
            pltpu.VMEM((span,), jnp.float32),
            pltpu.VMEM_SHARED((npad,), jnp.float32),
        ],
    )(dst2d, w2d)


@functools.partial(jax.jit, static_argnames=("npad", "span", "rows"))
def _run_sc_t(src2d, dst2d, w2d, u, *, npad, span, rows):
    mesh = plsc.VectorSubcoreMesh(core_axis_name="c", subcore_axis_name="s")
    body = functools.partial(_t_kernel_body, npad, span, rows)
    return pl.kernel(
        body,
        out_type=jax.ShapeDtypeStruct((NC * npad,), jnp.float32),
        mesh=mesh,
        compiler_params=pltpu.CompilerParams(needs_layout_passes=False),
        scratch_types=[
            pltpu.VMEM((R, LANES), jnp.int32),
            pltpu.VMEM((R, LANES), jnp.int32),
            pltpu.VMEM((R, LANES), jnp.float32),
            pltpu.VMEM((R, LANES), jnp.float32),
            pltpu.VMEM((span,), jnp.float32),
            pltpu.VMEM((npad,), jnp.float32),
            pltpu.VMEM_SHARED((npad,), jnp.float32),
        ],
    )(src2d, dst2d, w2d, u)


def kernel(x, edge_index, edge_weight, W_conv, b_conv,
           W1, b1, W2, b2, W3, b3, W4, b4, W5, b5):
    n = x.shape[0]
    e = edge_index.shape[1]
    assert e % LANES == 0
    rows = e // LANES
    span = -(-n // (NS * 32)) * 32
    npad = span * NS

    ei = edge_index.astype(jnp.int32)
    src2d = ei[0].reshape(rows, LANES)
    dst2d = ei[1].reshape(rows, LANES)
    w2d = edge_weight.astype(jnp.float32).reshape(rows, LANES)

    deg_p = _run_sc_deg(dst2d, w2d, npad=npad, span=span, rows=rows).reshape(NC, npad)
    deg = deg_p[0] + deg_p[1] + 1.0
    dinv = jnp.where(deg > 0, lax.rsqrt(jnp.where(deg > 0, deg, 1.0)), 0.0)
    xf = jnp.pad(x[:, 0].astype(jnp.float32), (0, npad - n))
    u = xf * dinv

    t_p = _run_sc_t(src2d, dst2d, w2d, u, npad=npad, span=span,
                    rows=rows).reshape(NC, npad)

    sb = dinv * (t_p[0] + t_p[1]) + dinv * dinv * xf
    zin = jnp.concatenate([sb[:, None], xf[:, None]], axis=1)
    wc64 = jnp.concatenate([W_conv[0], jnp.zeros((1,), jnp.float32)]).reshape(1, 64)
    bc64 = jnp.concatenate([b_conv, jnp.zeros((1,), jnp.float32)]).reshape(1, 64)
    e64 = jnp.zeros((1, 64), jnp.float32).at[0, 63].set(1.0)

    bd = 7168
    grid = npad // bd
    assert npad % bd == 0
    w64_spec = pl.BlockSpec((64, 64), lambda i: (0, 0))
    row_spec = pl.BlockSpec((1, 64), lambda i: (0, 0))

    out_pad = pl.pallas_call(
        _mlp_kernel_body,
        grid=(grid,),
        in_specs=[pl.BlockSpec((bd, 2), lambda i: (i, 0)),
                  row_spec, row_spec, row_spec,
                  w64_spec, row_spec, w64_spec, row_spec, w64_spec, row_spec,
                  w64_spec, row_spec,
                  pl.BlockSpec((64, 1), lambda i: (0, 0)),
                  pl.BlockSpec((1, 1), lambda i: (0, 0))],
        out_specs=pl.BlockSpec((bd, 1), lambda i: (i, 0)),
        out_shape=jax.ShapeDtypeStruct((npad, 1), jnp.float32),
    )(zin, wc64, bc64, e64, W1, b1.reshape(1, 64), W2, b2.reshape(1, 64),
      W3, b3.reshape(1, 64), W4, b4.reshape(1, 64), W5, b5.reshape(1, 1))

    return out_pad[:n]


# speedup vs baseline: 351.4558x; 2.3983x over previous
import functools

import jax
import jax.numpy as jnp
from jax import lax
from jax.experimental import pallas as pl
from jax.experimental.pallas import tpu as pltpu
from jax.experimental.pallas import tpu_sc as plsc

NC = 2
NS = 16
LANES = 128
RC = 8


def _row_split(rows, w):
    nw = NC * NS
    blocks = rows // 8
    base = blocks // nw
    extra = blocks % nw
    start = 8 * (w * base + jnp.minimum(w, extra))
    nrows = 8 * (base + (w < extra).astype(jnp.int32))
    return start, nrows


def _deg_kernel_body(npad, span, rows, dst_hbm, w_hbm, out_hbm,
                     dbuf, wbuf, zbuf, deg_sh,
                     si0, si1, si2, si3, ss0, ss1, ss2, ss3):
    c = lax.axis_index("c")
    s = lax.axis_index("s")
    w = c * NS + s
    sin = (si0, si1, si2, si3)
    ssc = (ss0, ss1, ss2, ss3)

    def zb(i, _):
        zbuf[pl.ds(i * 16, 16)] = jnp.zeros((16,), jnp.float32)
        return 0
    lax.fori_loop(0, span // 16, zb, 0)
    pltpu.sync_copy(zbuf, deg_sh.at[pl.ds(s * span, span)])
    plsc.subcore_barrier()

    start, nrows = _row_split(rows, w)
    nchunk = nrows // RC

    def start_in(k, b):
        r0 = start + k * RC
        pltpu.async_copy(dst_hbm.at[pl.ds(r0, RC)], dbuf.at[b], sin[b])
        pltpu.async_copy(w_hbm.at[pl.ds(r0, RC)], wbuf.at[b], sin[b])

    def wait_in(b):
        pltpu.make_async_copy(dst_hbm.at[pl.ds(0, RC)], dbuf.at[b], sin[b]).wait()
        pltpu.make_async_copy(w_hbm.at[pl.ds(0, RC)], wbuf.at[b], sin[b]).wait()

    def fire_sc(b):
        for i in range(RC):
            pltpu.async_copy(wbuf.at[b, i], deg_sh.at[dbuf.at[b, i]], ssc[b],
                             add=True)

    def drain_sc(b):
        for i in range(RC):
            pltpu.make_async_copy(wbuf.at[b, i], deg_sh.at[dbuf.at[b, i]],
                                  ssc[b]).wait()

    start_in(0, 0)
    start_in(1, 1)

    def body(k4, _):
        for b in range(4):
            k = k4 * 4 + b
            bn = (b + 2) % 4

            @pl.when((k >= 2) & (k - 2 < nchunk))
            def _():
                drain_sc(bn)

            @pl.when(k + 2 < nchunk)
            def _():
                start_in(k + 2, bn)

            @pl.when(k < nchunk)
            def _():
                wait_in(b)
                fire_sc(b)
        return 0
    lax.fori_loop(0, (nchunk + 5) // 4, body, 0)

    plsc.subcore_barrier()
    pltpu.sync_copy(deg_sh.at[pl.ds(s * span, span)],
                    out_hbm.at[pl.ds(c * npad + s * span, span)])


def _t_kernel_body(npad, span, rows, src_hbm, dst_hbm, w_hbm, u_hbm, out_hbm,
                   sbuf, dbuf, wbuf, pbuf, zbuf, u_v, t_sh,
                   si0, si1, si2, si3, ss0, ss1, ss2, ss3):
    c = lax.axis_index("c")
    s = lax.axis_index("s")
    w = c * NS + s
    sin = (si0, si1, si2, si3)
    ssc = (ss0, ss1, ss2, ss3)

    def zb(i, _):
        zbuf[pl.ds(i * 16, 16)] = jnp.zeros((16,), jnp.float32)
        return 0
    lax.fori_loop(0, span // 16, zb, 0)
    pltpu.sync_copy(zbuf, t_sh.at[pl.ds(s * span, span)])
    pltpu.sync_copy(u_hbm, u_v)
    plsc.subcore_barrier()

    start, nrows = _row_split(rows, w)
    nchunk = nrows // RC

    def start_in(k, b):
        r0 = start + k * RC
        pltpu.async_copy(src_hbm.at[pl.ds(r0, RC)], sbuf.at[b], sin[b])
        pltpu.async_copy(dst_hbm.at[pl.ds(r0, RC)], dbuf.at[b], sin[b])
        pltpu.async_copy(w_hbm.at[pl.ds(r0, RC)], wbuf.at[b], sin[b])

    def wait_in(b):
        pltpu.make_async_copy(src_hbm.at[pl.ds(0, RC)], sbuf.at[b], sin[b]).wait()
        pltpu.make_async_copy(dst_hbm.at[pl.ds(0, RC)], dbuf.at[b], sin[b]).wait()
        pltpu.make_async_copy(w_hbm.at[pl.ds(0, RC)], wbuf.at[b], sin[b]).wait()

    def compute(b):
        for i in range(RC):
            for j in range(LANES // 16):
                sl = pl.ds(j * 16, 16)
                g = plsc.load_gather(u_v, [sbuf[b, i, sl]])
                pbuf[b, i, sl] = g * wbuf[b, i, sl]

    def fire_sc(b):
        for i in range(RC):
            pltpu.async_copy(pbuf.at[b, i], t_sh.at[dbuf.at[b, i]], ssc[b],
                             add=True)

    def drain_sc(b):
        for i in range(RC):
            pltpu.make_async_copy(pbuf.at[b, i], t_sh.at[dbuf.at[b, i]],
                                  ssc[b]).wait()

    start_in(0, 0)
    start_in(1, 1)

    def body(k4, _):
        for b in range(4):
            k = k4 * 4 + b
            bn = (b + 2) % 4

            @pl.when((k >= 2) & (k - 2 < nchunk))
            def _():
                drain_sc(bn)

            @pl.when(k + 2 < nchunk)
            def _():
                start_in(k + 2, bn)

            @pl.when(k < nchunk)
            def _():
                wait_in(b)
                compute(b)
                fire_sc(b)
        return 0
    lax.fori_loop(0, (nchunk + 5) // 4, body, 0)

    plsc.subcore_barrier()
    pltpu.sync_copy(t_sh.at[pl.ds(s * span, span)],
                    out_hbm.at[pl.ds(c * npad + s * span, span)])


def _mlp_kernel_body(zr, wc64r, bc64r, e64r,
                     w1r, b1r, w2r, b2r, w3r, b3r, w4r, b4r, w5r, b5r, outr):
    sb = zr[..., 0:1]
    xb = zr[..., 1:2]
    z = sb * wc64r[...] + xb * e64r[...] + bc64r[...]
    h = jnp.maximum(jnp.dot(z, w1r[...]) + b1r[...], 0.0)
    h = jnp.maximum(jnp.dot(h, w2r[...]) + b2r[...], 0.0)
    h = jnp.maximum(jnp.dot(h, w3r[...]) + b3r[...], 0.0)
    h = jnp.maximum(jnp.dot(h, w4r[...]) + b4r[...], 0.0)
    outr[...] = jnp.dot(h, w5r[...]) + b5r[...]


@functools.partial(jax.jit, static_argnames=("npad", "span", "rows"))
def _run_sc_deg(dst2d, w2d, *, npad, span, rows):
    mesh = plsc.VectorSubcoreMesh(core_axis_name="c", subcore_axis_name="s")
    body = functools.partial(_deg_kernel_body, npad, span, rows)
    return pl.kernel(
        body,
        out_type=jax.ShapeDtypeStruct((NC * npad,), jnp.float32),
        mesh=mesh,
        compiler_params=pltpu.CompilerParams(needs_layout_passes=False),
        scratch_types=[
            pltpu.VMEM((4, RC, LANES), jnp.int32),
            pltpu.VMEM((4, RC, LANES), jnp.float32),
            pltpu.VMEM((span,), jnp.float32),
            pltpu.VMEM_SHARED((npad,), jnp.float32),
            pltpu.SemaphoreType.DMA, pltpu.SemaphoreType.DMA,
            pltpu.SemaphoreType.DMA, pltpu.SemaphoreType.DMA,
            pltpu.SemaphoreType.DMA, pltpu.SemaphoreType.DMA,
            pltpu.SemaphoreType.DMA, pltpu.SemaphoreType.DMA,
        ],
    )(dst2d, w2d)


@functools.partial(jax.jit, static_argnames=("npad", "span", "rows"))
def _run_sc_t(src2d, dst2d, w2d, u, *, npad, span, rows):
    mesh = plsc.VectorSubcoreMesh(core_axis_name="c", subcore_axis_name="s")
    body = functools.partial(_t_kernel_body, npad, span, rows)
    return pl.kernel(
        body,
        out_type=jax.ShapeDtypeStruct((NC * npad,), jnp.float32),
        mesh=mesh,
        compiler_params=pltpu.CompilerParams(needs_layout_passes=False),
        scratch_types=[
            pltpu.VMEM((4, RC, LANES), jnp.int32),
            pltpu.VMEM((4, RC, LANES), jnp.int32),
            pltpu.VMEM((4, RC, LANES), jnp.float32),
            pltpu.VMEM((4, RC, LANES), jnp.float32),
            pltpu.VMEM((span,), jnp.float32),
            pltpu.VMEM((npad,), jnp.float32),
            pltpu.VMEM_SHARED((npad,), jnp.float32),
            pltpu.SemaphoreType.DMA, pltpu.SemaphoreType.DMA,
            pltpu.SemaphoreType.DMA, pltpu.SemaphoreType.DMA,
            pltpu.SemaphoreType.DMA, pltpu.SemaphoreType.DMA,
            pltpu.SemaphoreType.DMA, pltpu.SemaphoreType.DMA,
        ],
    )(src2d, dst2d, w2d, u)


def kernel(x, edge_index, edge_weight, W_conv, b_conv,
           W1, b1, W2, b2, W3, b3, W4, b4, W5, b5):
    n = x.shape[0]
    e = edge_index.shape[1]
    assert e % LANES == 0
    rows = e // LANES
    span = -(-n // (NS * 32)) * 32
    npad = span * NS

    ei = edge_index.astype(jnp.int32)
    src2d = ei[0].reshape(rows, LANES)
    dst2d = ei[1].reshape(rows, LANES)
    w2d = edge_weight.astype(jnp.float32).reshape(rows, LANES)

    deg_p = _run_sc_deg(dst2d, w2d, npad=npad, span=span, rows=rows).reshape(NC, npad)
    deg = deg_p[0] + deg_p[1] + 1.0
    dinv = jnp.where(deg > 0, lax.rsqrt(jnp.where(deg > 0, deg, 1.0)), 0.0)
    xf = jnp.pad(x[:, 0].astype(jnp.float32), (0, npad - n))
    u = xf * dinv

    t_p = _run_sc_t(src2d, dst2d, w2d, u, npad=npad, span=span,
                    rows=rows).reshape(NC, npad)

    sb = dinv * (t_p[0] + t_p[1]) + dinv * dinv * xf
    zin = jnp.concatenate([sb[:, None], xf[:, None]], axis=1)
    wc64 = jnp.concatenate([W_conv[0], jnp.zeros((1,), jnp.float32)]).reshape(1, 64)
    bc64 = jnp.concatenate([b_conv, jnp.zeros((1,), jnp.float32)]).reshape(1, 64)
    e64 = jnp.zeros((1, 64), jnp.float32).at[0, 63].set(1.0)

    bd = 7168
    grid = npad // bd
    assert npad % bd == 0
    w64_spec = pl.BlockSpec((64, 64), lambda i: (0, 0))
    row_spec = pl.BlockSpec((1, 64), lambda i: (0, 0))

    out_pad = pl.pallas_call(
        _mlp_kernel_body,
        grid=(grid,),
        in_specs=[pl.BlockSpec((bd, 2), lambda i: (i, 0)),
                  row_spec, row_spec, row_spec,
                  w64_spec, row_spec, w64_spec, row_spec, w64_spec, row_spec,
                  w64_spec, row_spec,
                  pl.BlockSpec((64, 1), lambda i: (0, 0)),
                  pl.BlockSpec((1, 1), lambda i: (0, 0))],
        out_specs=pl.BlockSpec((bd, 1), lambda i: (i, 0)),
        out_shape=jax.ShapeDtypeStruct((npad, 1), jnp.float32),
    )(zin, wc64, bc64, e64, W1, b1.reshape(1, 64), W2, b2.reshape(1, 64),
      W3, b3.reshape(1, 64), W4, b4.reshape(1, 64), W5, b5.reshape(1, 1))

    return out_pad[:n]


# speedup vs baseline: 502.0693x; 1.4285x over previous
import functools

import jax
import jax.numpy as jnp
from jax import lax
from jax.experimental import pallas as pl
from jax.experimental.pallas import tpu as pltpu
from jax.experimental.pallas import tpu_sc as plsc

NC = 2
NS = 16
LANES = 128
RC = 8


def _row_split(rows, w):
    nw = NC * NS
    blocks = rows // 8
    base = blocks // nw
    extra = blocks % nw
    start = 8 * (w * base + jnp.minimum(w, extra))
    nrows = 8 * (base + (w < extra).astype(jnp.int32))
    return start, nrows


def _deg_kernel_body(npad, span, rows, dst_hbm, w_hbm, out_hbm,
                     dbuf, wbuf, zbuf, deg_sh,
                     si0, si1, si2, si3, ss0, ss1, ss2, ss3):
    c = lax.axis_index("c")
    s = lax.axis_index("s")
    w = c * NS + s
    sin = (si0, si1, si2, si3)
    ssc = (ss0, ss1, ss2, ss3)

    def zb(i, _):
        zbuf[pl.ds(i * 16, 16)] = jnp.zeros((16,), jnp.float32)
        return 0
    lax.fori_loop(0, span // 16, zb, 0)
    pltpu.sync_copy(zbuf, deg_sh.at[pl.ds(s * span, span)])
    plsc.subcore_barrier()

    start, nrows = _row_split(rows, w)
    nchunk = nrows // RC

    def start_in(k, b):
        r0 = start + k * RC
        pltpu.async_copy(dst_hbm.at[pl.ds(r0, RC)], dbuf.at[b], sin[b])
        pltpu.async_copy(w_hbm.at[pl.ds(r0, RC)], wbuf.at[b], sin[b])

    def wait_in(b):
        pltpu.make_async_copy(dst_hbm.at[pl.ds(0, RC)], dbuf.at[b], sin[b]).wait()
        pltpu.make_async_copy(w_hbm.at[pl.ds(0, RC)], wbuf.at[b], sin[b]).wait()

    def fire_sc(b):
        for i in range(RC):
            pltpu.async_copy(wbuf.at[b, i], deg_sh.at[dbuf.at[b, i]], ssc[b],
                             add=True)

    def drain_sc(b):
        for i in range(RC):
            pltpu.make_async_copy(wbuf.at[b, i], deg_sh.at[dbuf.at[b, i]],
                                  ssc[b]).wait()

    start_in(0, 0)
    start_in(1, 1)

    def body(k4, _):
        for b in range(4):
            k = k4 * 4 + b
            bn = (b + 2) % 4

            @pl.when((k >= 2) & (k - 2 < nchunk))
            def _():
                drain_sc(bn)

            @pl.when(k + 2 < nchunk)
            def _():
                start_in(k + 2, bn)

            @pl.when(k < nchunk)
            def _():
                wait_in(b)
                fire_sc(b)
        return 0
    lax.fori_loop(0, (nchunk + 5) // 4, body, 0)

    plsc.subcore_barrier()
    pltpu.sync_copy(deg_sh.at[pl.ds(s * span, span)],
                    out_hbm.at[pl.ds(c * npad + s * span, span)])


def _t_kernel_body(npad, span, rows, src_hbm, dst_hbm, w_hbm, u_hbm, out_hbm,
                   sbuf, dbuf, wbuf, pbuf, zbuf, u_v, t_sh,
                   si0, si1, si2, si3, ss0, ss1, ss2, ss3):
    c = lax.axis_index("c")
    s = lax.axis_index("s")
    w = c * NS + s
    sin = (si0, si1, si2, si3)
    ssc = (ss0, ss1, ss2, ss3)

    def zb(i, _):
        zbuf[pl.ds(i * 16, 16)] = jnp.zeros((16,), jnp.float32)
        return 0
    lax.fori_loop(0, span // 16, zb, 0)
    pltpu.sync_copy(zbuf, t_sh.at[pl.ds(s * span, span)])
    pltpu.sync_copy(u_hbm, u_v)
    plsc.subcore_barrier()

    start, nrows = _row_split(rows, w)
    nchunk = nrows // RC

    def start_in(k, b):
        r0 = start + k * RC
        pltpu.async_copy(src_hbm.at[pl.ds(r0, RC)], sbuf.at[b], sin[b])
        pltpu.async_copy(dst_hbm.at[pl.ds(r0, RC)], dbuf.at[b], sin[b])
        pltpu.async_copy(w_hbm.at[pl.ds(r0, RC)], wbuf.at[b], sin[b])

    def wait_in(b):
        pltpu.make_async_copy(src_hbm.at[pl.ds(0, RC)], sbuf.at[b], sin[b]).wait()
        pltpu.make_async_copy(dst_hbm.at[pl.ds(0, RC)], dbuf.at[b], sin[b]).wait()
        pltpu.make_async_copy(w_hbm.at[pl.ds(0, RC)], wbuf.at[b], sin[b]).wait()

    def compute(b):
        for i in range(RC):
            for j in range(LANES // 16):
                sl = pl.ds(j * 16, 16)
                g = plsc.load_gather(u_v, [sbuf[b, i, sl]])
                pbuf[b, i, sl] = g * wbuf[b, i, sl]

    def fire_sc(b):
        for i in range(RC):
            pltpu.async_copy(pbuf.at[b, i], t_sh.at[dbuf.at[b, i]], ssc[b],
                             add=True)

    def drain_sc(b):
        for i in range(RC):
            pltpu.make_async_copy(pbuf.at[b, i], t_sh.at[dbuf.at[b, i]],
                                  ssc[b]).wait()

    start_in(0, 0)
    start_in(1, 1)

    def body(k4, _):
        for b in range(4):
            k = k4 * 4 + b
            bn = (b + 2) % 4

            @pl.when((k >= 2) & (k - 2 < nchunk))
            def _():
                drain_sc(bn)

            @pl.when(k + 2 < nchunk)
            def _():
                start_in(k + 2, bn)

            @pl.when(k < nchunk)
            def _():
                wait_in(b)
                compute(b)
                fire_sc(b)
        return 0
    lax.fori_loop(0, (nchunk + 5) // 4, body, 0)

    plsc.subcore_barrier()
    pltpu.sync_copy(t_sh.at[pl.ds(s * span, span)],
                    out_hbm.at[pl.ds(c * npad + s * span, span)])


def _mlp_kernel_body(zr, wc64r, bc64r, e64r,
                     w1r, b1r, w2r, b2r, w3r, b3r, w4r, b4r, w5r, b5r, outr):
    sb = zr[0:1, :]
    xb = zr[1:2, :]
    z = wc64r[...] * sb + e64r[...] * xb + bc64r[...]
    h = jnp.maximum(jnp.dot(w1r[...], z) + b1r[...], 0.0)
    h = jnp.maximum(jnp.dot(w2r[...], h) + b2r[...], 0.0)
    h = jnp.maximum(jnp.dot(w3r[...], h) + b3r[...], 0.0)
    h = jnp.maximum(jnp.dot(w4r[...], h) + b4r[...], 0.0)
    outr[...] = jnp.dot(w5r[...], h) + b5r[...]


@functools.partial(jax.jit, static_argnames=("npad", "span", "rows"))
def _run_sc_deg(dst2d, w2d, *, npad, span, rows):
    mesh = plsc.VectorSubcoreMesh(core_axis_name="c", subcore_axis_name="s")
    body = functools.partial(_deg_kernel_body, npad, span, rows)
    return pl.kernel(
        body,
        out_type=jax.ShapeDtypeStruct((NC * npad,), jnp.float32),
        mesh=mesh,
        compiler_params=pltpu.CompilerParams(needs_layout_passes=False),
        scratch_types=[
            pltpu.VMEM((4, RC, LANES), jnp.int32),
            pltpu.VMEM((4, RC, LANES), jnp.float32),
            pltpu.VMEM((span,), jnp.float32),
            pltpu.VMEM_SHARED((npad,), jnp.float32),
            pltpu.SemaphoreType.DMA, pltpu.SemaphoreType.DMA,
            pltpu.SemaphoreType.DMA, pltpu.SemaphoreType.DMA,
            pltpu.SemaphoreType.DMA, pltpu.SemaphoreType.DMA,
            pltpu.SemaphoreType.DMA, pltpu.SemaphoreType.DMA,
        ],
    )(dst2d, w2d)


@functools.partial(jax.jit, static_argnames=("npad", "span", "rows"))
def _run_sc_t(src2d, dst2d, w2d, u, *, npad, span, rows):
    mesh = plsc.VectorSubcoreMesh(core_axis_name="c", subcore_axis_name="s")
    body = functools.partial(_t_kernel_body, npad, span, rows)
    return pl.kernel(
        body,
        out_type=jax.ShapeDtypeStruct((NC * npad,), jnp.float32),
        mesh=mesh,
        compiler_params=pltpu.CompilerParams(needs_layout_passes=False),
        scratch_types=[
            pltpu.VMEM((4, RC, LANES), jnp.int32),
            pltpu.VMEM((4, RC, LANES), jnp.int32),
            pltpu.VMEM((4, RC, LANES), jnp.float32),
            pltpu.VMEM((4, RC, LANES), jnp.float32),
            pltpu.VMEM((span,), jnp.float32),
            pltpu.VMEM((npad,), jnp.float32),
            pltpu.VMEM_SHARED((npad,), jnp.float32),
            pltpu.SemaphoreType.DMA, pltpu.SemaphoreType.DMA,
            pltpu.SemaphoreType.DMA, pltpu.SemaphoreType.DMA,
            pltpu.SemaphoreType.DMA, pltpu.SemaphoreType.DMA,
            pltpu.SemaphoreType.DMA, pltpu.SemaphoreType.DMA,
        ],
    )(src2d, dst2d, w2d, u)


def kernel(x, edge_index, edge_weight, W_conv, b_conv,
           W1, b1, W2, b2, W3, b3, W4, b4, W5, b5):
    n = x.shape[0]
    e = edge_index.shape[1]
    assert e % LANES == 0
    rows = e // LANES
    span = -(-n // (NS * 32)) * 32
    npad = span * NS

    ei = edge_index.astype(jnp.int32)
    src2d = ei[0].reshape(rows, LANES)
    dst2d = ei[1].reshape(rows, LANES)
    w2d = edge_weight.astype(jnp.float32).reshape(rows, LANES)

    deg_p = _run_sc_deg(dst2d, w2d, npad=npad, span=span, rows=rows).reshape(NC, npad)
    deg = deg_p[0] + deg_p[1] + 1.0
    dinv = jnp.where(deg > 0, lax.rsqrt(jnp.where(deg > 0, deg, 1.0)), 0.0)
    xf = jnp.pad(x[:, 0].astype(jnp.float32), (0, npad - n))
    u = xf * dinv

    t_p = _run_sc_t(src2d, dst2d, w2d, u, npad=npad, span=span,
                    rows=rows).reshape(NC, npad)

    sb = dinv * (t_p[0] + t_p[1]) + dinv * dinv * xf
    zt = jnp.stack([sb[:n], xf[:n]])
    wc64 = jnp.concatenate([W_conv[0], jnp.zeros((1,), jnp.float32)]).reshape(64, 1)
    bc64 = jnp.concatenate([b_conv, jnp.zeros((1,), jnp.float32)]).reshape(64, 1)
    e64 = jnp.zeros((64, 1), jnp.float32).at[63, 0].set(1.0)

    bd = 12544
    grid = -(-n // bd)
    w64_spec = pl.BlockSpec((64, 64), lambda i: (0, 0))
    col_spec = pl.BlockSpec((64, 1), lambda i: (0, 0))

    out_t = pl.pallas_call(
        _mlp_kernel_body,
        grid=(grid,),
        in_specs=[pl.BlockSpec((2, bd), lambda i: (0, i)),
                  col_spec, col_spec, col_spec,
                  w64_spec, col_spec, w64_spec, col_spec, w64_spec, col_spec,
                  w64_spec, col_spec,
                  pl.BlockSpec((1, 64), lambda i: (0, 0)),
                  pl.BlockSpec((1, 1), lambda i: (0, 0))],
        out_specs=pl.BlockSpec((1, bd), lambda i: (0, i)),
        out_shape=jax.ShapeDtypeStruct((1, n), jnp.float32),
    )(zt, wc64, bc64, e64, W1.T, b1.reshape(64, 1), W2.T, b2.reshape(64, 1),
      W3.T, b3.reshape(64, 1), W4.T, b4.reshape(64, 1), W5.T, b5.reshape(1, 1))

    return out_t.reshape(n, 1)


# speedup vs baseline: 544.7869x; 1.0851x over previous
import functools

import jax
import jax.numpy as jnp
from jax import lax
from jax.experimental import pallas as pl
from jax.experimental.pallas import tpu as pltpu
from jax.experimental.pallas import tpu_sc as plsc

NC = 2
NS = 16
LANES = 128
RC = 8
ECH = RC * LANES


def _row_split(rows, w):
    nw = NC * NS
    blocks = rows // 8
    base = blocks // nw
    extra = blocks % nw
    start = 8 * (w * base + jnp.minimum(w, extra))
    nrows = 8 * (base + (w < extra).astype(jnp.int32))
    return start, nrows


def _deg_kernel_body(npad, span, rows, ei_hbm, w_hbm, out_hbm,
                     db0, db1, db2, db3, wb0, wb1, wb2, wb3, zbuf, deg_sh,
                     si0, si1, si2, si3, ss0, ss1, ss2, ss3):
    dbufs = (db0, db1, db2, db3)
    wbufs = (wb0, wb1, wb2, wb3)
    c = lax.axis_index("c")
    s = lax.axis_index("s")
    w = c * NS + s
    sin = (si0, si1, si2, si3)
    ssc = (ss0, ss1, ss2, ss3)

    def zb(i, _):
        zbuf[pl.ds(i * 16, 16)] = jnp.zeros((16,), jnp.float32)
        return 0
    lax.fori_loop(0, span // 16, zb, 0)
    pltpu.sync_copy(zbuf, deg_sh.at[pl.ds(s * span, span)])
    plsc.subcore_barrier()

    start, nrows = _row_split(rows, w)
    nchunk = nrows // RC

    def start_in(k, b):
        e0 = (start + k * RC) * LANES
        pltpu.async_copy(ei_hbm.at[pl.ds(rows * LANES + e0, ECH)], dbufs[b],
                         sin[b])
        pltpu.async_copy(w_hbm.at[pl.ds(e0, ECH)], wbufs[b], sin[b])

    def wait_in(b):
        pltpu.make_async_copy(ei_hbm.at[pl.ds(0, ECH)], dbufs[b], sin[b]).wait()
        pltpu.make_async_copy(w_hbm.at[pl.ds(0, ECH)], wbufs[b], sin[b]).wait()

    def fire_sc(b):
        pltpu.async_copy(wbufs[b], deg_sh.at[dbufs[b]], ssc[b], add=True)

    def drain_sc(b):
        pltpu.make_async_copy(wbufs[b], deg_sh.at[dbufs[b]], ssc[b]).wait()

    start_in(0, 0)
    start_in(1, 1)

    def body(k4, _):
        for b in range(4):
            k = k4 * 4 + b
            bn = (b + 2) % 4

            @pl.when((k >= 2) & (k - 2 < nchunk))
            def _():
                drain_sc(bn)

            @pl.when(k + 2 < nchunk)
            def _():
                start_in(k + 2, bn)

            @pl.when(k < nchunk)
            def _():
                wait_in(b)
                fire_sc(b)
        return 0
    lax.fori_loop(0, (nchunk + 5) // 4, body, 0)

    plsc.subcore_barrier()
    pltpu.sync_copy(deg_sh.at[pl.ds(s * span, span)],
                    out_hbm.at[pl.ds(c * npad + s * span, span)])


def _t_kernel_body(npad, span, rows, ei_hbm, w_hbm, u_hbm, out_hbm,
                   sb0, sb1, sb2, sb3, db0, db1, db2, db3,
                   wb0, wb1, wb2, wb3, pb0, pb1, pb2, pb3, zbuf, u_v, t_sh,
                   si0, si1, si2, si3, ss0, ss1, ss2, ss3):
    sbufs = (sb0, sb1, sb2, sb3)
    dbufs = (db0, db1, db2, db3)
    wbufs = (wb0, wb1, wb2, wb3)
    pbufs = (pb0, pb1, pb2, pb3)
    c = lax.axis_index("c")
    s = lax.axis_index("s")
    w = c * NS + s
    sin = (si0, si1, si2, si3)
    ssc = (ss0, ss1, ss2, ss3)

    def zb(i, _):
        zbuf[pl.ds(i * 16, 16)] = jnp.zeros((16,), jnp.float32)
        return 0
    lax.fori_loop(0, span // 16, zb, 0)
    pltpu.sync_copy(zbuf, t_sh.at[pl.ds(s * span, span)])
    pltpu.sync_copy(u_hbm, u_v)
    plsc.subcore_barrier()

    start, nrows = _row_split(rows, w)
    nchunk = nrows // RC

    def start_in(k, b):
        e0 = (start + k * RC) * LANES
        pltpu.async_copy(ei_hbm.at[pl.ds(e0, ECH)], sbufs[b], sin[b])
        pltpu.async_copy(ei_hbm.at[pl.ds(rows * LANES + e0, ECH)], dbufs[b],
                         sin[b])
        pltpu.async_copy(w_hbm.at[pl.ds(e0, ECH)], wbufs[b], sin[b])

    def wait_in(b):
        pltpu.make_async_copy(ei_hbm.at[pl.ds(0, ECH)], sbufs[b], sin[b]).wait()
        pltpu.make_async_copy(ei_hbm.at[pl.ds(0, ECH)], dbufs[b], sin[b]).wait()
        pltpu.make_async_copy(w_hbm.at[pl.ds(0, ECH)], wbufs[b], sin[b]).wait()

    def compute(b):
        for g in range(ECH // 16):
            sl = pl.ds(g * 16, 16)
            gv = plsc.load_gather(u_v, [sbufs[b][sl]])
            pbufs[b][sl] = gv * wbufs[b][sl]

    def fire_sc(b):
        pltpu.async_copy(pbufs[b], t_sh.at[dbufs[b]], ssc[b], add=True)

    def drain_sc(b):
        pltpu.make_async_copy(pbufs[b], t_sh.at[dbufs[b]], ssc[b]).wait()

    start_in(0, 0)
    start_in(1, 1)

    def body(k4, _):
        for b in range(4):
            k = k4 * 4 + b
            bn = (b + 2) % 4

            @pl.when((k >= 2) & (k - 2 < nchunk))
            def _():
                drain_sc(bn)

            @pl.when(k + 2 < nchunk)
            def _():
                start_in(k + 2, bn)

            @pl.when(k < nchunk)
            def _():
                wait_in(b)
                compute(b)
                fire_sc(b)
        return 0
    lax.fori_loop(0, (nchunk + 5) // 4, body, 0)

    plsc.subcore_barrier()
    pltpu.sync_copy(t_sh.at[pl.ds(s * span, span)],
                    out_hbm.at[pl.ds(c * npad + s * span, span)])


def _mlp_kernel_body(zr, wc64r, bc64r, e64r,
                     w1r, b1r, w2r, b2r, w3r, b3r, w4r, b4r, w5r, b5r, outr):
    sb = zr[0:1, :]
    xb = zr[1:2, :]
    z = wc64r[...] * sb + e64r[...] * xb + bc64r[...]
    h = jnp.maximum(jnp.dot(w1r[...], z) + b1r[...], 0.0)
    h = jnp.maximum(jnp.dot(w2r[...], h) + b2r[...], 0.0)
    h = jnp.maximum(jnp.dot(w3r[...], h) + b3r[...], 0.0)
    h = jnp.maximum(jnp.dot(w4r[...], h) + b4r[...], 0.0)
    outr[...] = jnp.dot(w5r[...], h) + b5r[...]


@functools.partial(jax.jit, static_argnames=("npad", "span", "rows"))
def _run_sc_deg(ei1, w1, *, npad, span, rows):
    mesh = plsc.VectorSubcoreMesh(core_axis_name="c", subcore_axis_name="s")
    body = functools.partial(_deg_kernel_body, npad, span, rows)
    return pl.kernel(
        body,
        out_type=jax.ShapeDtypeStruct((NC * npad,), jnp.float32),
        mesh=mesh,
        compiler_params=pltpu.CompilerParams(needs_layout_passes=False),
        scratch_types=[
            pltpu.VMEM((ECH,), jnp.int32), pltpu.VMEM((ECH,), jnp.int32),
            pltpu.VMEM((ECH,), jnp.int32), pltpu.VMEM((ECH,), jnp.int32),
            pltpu.VMEM((ECH,), jnp.float32), pltpu.VMEM((ECH,), jnp.float32),
            pltpu.VMEM((ECH,), jnp.float32), pltpu.VMEM((ECH,), jnp.float32),
            pltpu.VMEM((span,), jnp.float32),
            pltpu.VMEM_SHARED((npad,), jnp.float32),
            pltpu.SemaphoreType.DMA, pltpu.SemaphoreType.DMA,
            pltpu.SemaphoreType.DMA, pltpu.SemaphoreType.DMA,
            pltpu.SemaphoreType.DMA, pltpu.SemaphoreType.DMA,
            pltpu.SemaphoreType.DMA, pltpu.SemaphoreType.DMA,
        ],
    )(ei1, w1)


@functools.partial(jax.jit, static_argnames=("npad", "span", "rows"))
def _run_sc_t(ei1, w1, u, *, npad, span, rows):
    mesh = plsc.VectorSubcoreMesh(core_axis_name="c", subcore_axis_name="s")
    body = functools.partial(_t_kernel_body, npad, span, rows)
    return pl.kernel(
        body,
        out_type=jax.ShapeDtypeStruct((NC * npad,), jnp.float32),
        mesh=mesh,
        compiler_params=pltpu.CompilerParams(needs_layout_passes=False),
        scratch_types=[
            pltpu.VMEM((ECH,), jnp.int32), pltpu.VMEM((ECH,), jnp.int32),
            pltpu.VMEM((ECH,), jnp.int32), pltpu.VMEM((ECH,), jnp.int32),
            pltpu.VMEM((ECH,), jnp.int32), pltpu.VMEM((ECH,), jnp.int32),
            pltpu.VMEM((ECH,), jnp.int32), pltpu.VMEM((ECH,), jnp.int32),
            pltpu.VMEM((ECH,), jnp.float32), pltpu.VMEM((ECH,), jnp.float32),
            pltpu.VMEM((ECH,), jnp.float32), pltpu.VMEM((ECH,), jnp.float32),
            pltpu.VMEM((ECH,), jnp.float32), pltpu.VMEM((ECH,), jnp.float32),
            pltpu.VMEM((ECH,), jnp.float32), pltpu.VMEM((ECH,), jnp.float32),
            pltpu.VMEM((span,), jnp.float32),
            pltpu.VMEM((npad,), jnp.float32),
            pltpu.VMEM_SHARED((npad,), jnp.float32),
            pltpu.SemaphoreType.DMA, pltpu.SemaphoreType.DMA,
            pltpu.SemaphoreType.DMA, pltpu.SemaphoreType.DMA,
            pltpu.SemaphoreType.DMA, pltpu.SemaphoreType.DMA,
            pltpu.SemaphoreType.DMA, pltpu.SemaphoreType.DMA,
        ],
    )(ei1, w1, u)


def kernel(x, edge_index, edge_weight, W_conv, b_conv,
           W1, b1, W2, b2, W3, b3, W4, b4, W5, b5):
    n = x.shape[0]
    e = edge_index.shape[1]
    assert e % LANES == 0
    rows = e // LANES
    span = -(-n // (NS * 32)) * 32
    npad = span * NS

    ei1 = edge_index.astype(jnp.int32).reshape(2 * rows * LANES)
    w1 = edge_weight.astype(jnp.float32)

    deg_p = _run_sc_deg(ei1, w1, npad=npad, span=span, rows=rows).reshape(NC, npad)
    deg = deg_p[0] + deg_p[1] + 1.0
    dinv = jnp.where(deg > 0, lax.rsqrt(jnp.where(deg > 0, deg, 1.0)), 0.0)
    xf = jnp.pad(x[:, 0].astype(jnp.float32), (0, npad - n))
    u = xf * dinv

    t_p = _run_sc_t(ei1, w1, u, npad=npad, span=span,
                    rows=rows).reshape(NC, npad)

    sb = dinv * (t_p[0] + t_p[1]) + dinv * dinv * xf
    zt = jnp.stack([sb[:n], xf[:n]])
    wc64 = jnp.concatenate([W_conv[0], jnp.zeros((1,), jnp.float32)]).reshape(64, 1)
    bc64 = jnp.concatenate([b_conv, jnp.zeros((1,), jnp.float32)]).reshape(64, 1)
    e64 = jnp.zeros((64, 1), jnp.float32).at[63, 0].set(1.0)

    bd = 12544
    grid = -(-n // bd)
    w64_spec = pl.BlockSpec((64, 64), lambda i: (0, 0))
    col_spec = pl.BlockSpec((64, 1), lambda i: (0, 0))

    out_t = pl.pallas_call(
        _mlp_kernel_body,
        grid=(grid,),
        in_specs=[pl.BlockSpec((2, bd), lambda i: (0, i)),
                  col_spec, col_spec, col_spec,
                  w64_spec, col_spec, w64_spec, col_spec, w64_spec, col_spec,
                  w64_spec, col_spec,
                  pl.BlockSpec((1, 64), lambda i: (0, 0)),
                  pl.BlockSpec((1, 1), lambda i: (0, 0))],
        out_specs=pl.BlockSpec((1, bd), lambda i: (0, i)),
        out_shape=jax.ShapeDtypeStruct((1, n), jnp.float32),
    )(zt, wc64, bc64, e64, W1.T, b1.reshape(64, 1), W2.T, b2.reshape(64, 1),
      W3.T, b3.reshape(64, 1), W4.T, b4.reshape(64, 1), W5.T, b5.reshape(1, 1))

    return out_t.reshape(n, 1)


# speedup vs baseline: 554.9432x; 1.0186x over previous
import functools

import jax
import jax.numpy as jnp
from jax import lax
from jax.experimental import pallas as pl
from jax.experimental.pallas import tpu as pltpu
from jax.experimental.pallas import tpu_sc as plsc

NC = 2
NS = 16
LANES = 128
RCD = 16
ECHD = RCD * LANES
RCT = 8
ECHT = RCT * LANES


def _row_split(rows, w, rc):
    nw = NC * NS
    blocks = rows // rc
    base = blocks // nw
    extra = blocks % nw
    start = rc * (w * base + jnp.minimum(w, extra))
    nrows = rc * (base + (w < extra).astype(jnp.int32))
    return start, nrows


def _deg_kernel_body(npad, span, rows, ei_hbm, w_hbm, out_hbm,
                     db0, db1, db2, db3, wb0, wb1, wb2, wb3, zbuf, deg_sh,
                     si0, si1, si2, si3, ss0, ss1, ss2, ss3):
    dbufs = (db0, db1, db2, db3)
    wbufs = (wb0, wb1, wb2, wb3)
    c = lax.axis_index("c")
    s = lax.axis_index("s")
    w = c * NS + s
    sin = (si0, si1, si2, si3)
    ssc = (ss0, ss1, ss2, ss3)

    def zb(i, _):
        zbuf[pl.ds(i * 16, 16)] = jnp.zeros((16,), jnp.float32)
        return 0
    lax.fori_loop(0, span // 16, zb, 0)
    pltpu.sync_copy(zbuf, deg_sh.at[pl.ds(s * span, span)])
    plsc.subcore_barrier()

    start, nrows = _row_split(rows, w, RCD)
    nchunk = nrows // RCD

    def start_in(k, b):
        e0 = (start + k * RCD) * LANES
        pltpu.async_copy(ei_hbm.at[pl.ds(rows * LANES + e0, ECHD)], dbufs[b],
                         sin[b])
        pltpu.async_copy(w_hbm.at[pl.ds(e0, ECHD)], wbufs[b], sin[b])

    def wait_in(b):
        pltpu.make_async_copy(ei_hbm.at[pl.ds(0, ECHD)], dbufs[b], sin[b]).wait()
        pltpu.make_async_copy(w_hbm.at[pl.ds(0, ECHD)], wbufs[b], sin[b]).wait()

    def fire_sc(b):
        pltpu.async_copy(wbufs[b], deg_sh.at[dbufs[b]], ssc[b], add=True)

    def drain_sc(b):
        pltpu.make_async_copy(wbufs[b], deg_sh.at[dbufs[b]], ssc[b]).wait()

    start_in(0, 0)
    start_in(1, 1)

    def body(k4, _):
        for b in range(4):
            k = k4 * 4 + b
            bn = (b + 2) % 4

            @pl.when((k >= 2) & (k - 2 < nchunk))
            def _():
                drain_sc(bn)

            @pl.when(k + 2 < nchunk)
            def _():
                start_in(k + 2, bn)

            @pl.when(k < nchunk)
            def _():
                wait_in(b)
                fire_sc(b)
        return 0
    lax.fori_loop(0, (nchunk + 5) // 4, body, 0)

    plsc.subcore_barrier()
    pltpu.sync_copy(deg_sh.at[pl.ds(s * span, span)],
                    out_hbm.at[pl.ds(c * npad + s * span, span)])


def _t_kernel_body(npad, span, rows, ei_hbm, w_hbm, u_hbm, out_hbm,
                   sb0, sb1, sb2, sb3, db0, db1, db2, db3,
                   wb0, wb1, wb2, wb3, pb0, pb1, pb2, pb3, zbuf, u_v, t_sh,
                   si0, si1, si2, si3, ss0, ss1, ss2, ss3):
    sbufs = (sb0, sb1, sb2, sb3)
    dbufs = (db0, db1, db2, db3)
    wbufs = (wb0, wb1, wb2, wb3)
    pbufs = (pb0, pb1, pb2, pb3)
    c = lax.axis_index("c")
    s = lax.axis_index("s")
    w = c * NS + s
    sin = (si0, si1, si2, si3)
    ssc = (ss0, ss1, ss2, ss3)

    def zb(i, _):
        zbuf[pl.ds(i * 16, 16)] = jnp.zeros((16,), jnp.float32)
        return 0
    lax.fori_loop(0, span // 16, zb, 0)
    pltpu.sync_copy(zbuf, t_sh.at[pl.ds(s * span, span)])
    pltpu.sync_copy(u_hbm, u_v)
    plsc.subcore_barrier()

    start, nrows = _row_split(rows, w, RCT)
    nchunk = nrows // RCT

    def start_in(k, b):
        e0 = (start + k * RCT) * LANES
        pltpu.async_copy(ei_hbm.at[pl.ds(e0, ECHT)], sbufs[b], sin[b])
        pltpu.async_copy(ei_hbm.at[pl.ds(rows * LANES + e0, ECHT)], dbufs[b],
                         sin[b])
        pltpu.async_copy(w_hbm.at[pl.ds(e0, ECHT)], wbufs[b], sin[b])

    def wait_in(b):
        pltpu.make_async_copy(ei_hbm.at[pl.ds(0, ECHT)], sbufs[b], sin[b]).wait()
        pltpu.make_async_copy(ei_hbm.at[pl.ds(0, ECHT)], dbufs[b], sin[b]).wait()
        pltpu.make_async_copy(w_hbm.at[pl.ds(0, ECHT)], wbufs[b], sin[b]).wait()

    def compute(b):
        for g in range(ECHT // 16):
            sl = pl.ds(g * 16, 16)
            gv = plsc.load_gather(u_v, [sbufs[b][sl]])
            pbufs[b][sl] = gv * wbufs[b][sl]

    def fire_sc(b):
        pltpu.async_copy(pbufs[b], t_sh.at[dbufs[b]], ssc[b], add=True)

    def drain_sc(b):
        pltpu.make_async_copy(pbufs[b], t_sh.at[dbufs[b]], ssc[b]).wait()

    start_in(0, 0)
    start_in(1, 1)

    def body(k4, _):
        for b in range(4):
            k = k4 * 4 + b
            bn = (b + 2) % 4

            @pl.when((k >= 2) & (k - 2 < nchunk))
            def _():
                drain_sc(bn)

            @pl.when(k + 2 < nchunk)
            def _():
                start_in(k + 2, bn)

            @pl.when(k < nchunk)
            def _():
                wait_in(b)
                compute(b)
                fire_sc(b)
        return 0
    lax.fori_loop(0, (nchunk + 5) // 4, body, 0)

    plsc.subcore_barrier()
    pltpu.sync_copy(t_sh.at[pl.ds(s * span, span)],
                    out_hbm.at[pl.ds(c * npad + s * span, span)])


def _mlp_kernel_body(zr, wc64r, bc64r, e64r,
                     w1r, b1r, w2r, b2r, w3r, b3r, w4r, b4r, w5r, b5r, outr):
    sb = zr[0:1, :]
    xb = zr[1:2, :]
    z = wc64r[...] * sb + e64r[...] * xb + bc64r[...]
    h = jnp.maximum(jnp.dot(w1r[...], z) + b1r[...], 0.0)
    h = jnp.maximum(jnp.dot(w2r[...], h) + b2r[...], 0.0)
    h = jnp.maximum(jnp.dot(w3r[...], h) + b3r[...], 0.0)
    h = jnp.maximum(jnp.dot(w4r[...], h) + b4r[...], 0.0)
    outr[...] = jnp.dot(w5r[...], h) + b5r[...]


@functools.partial(jax.jit, static_argnames=("npad", "span", "rows"))
def _run_sc_deg(ei1, w1, *, npad, span, rows):
    mesh = plsc.VectorSubcoreMesh(core_axis_name="c", subcore_axis_name="s")
    body = functools.partial(_deg_kernel_body, npad, span, rows)
    return pl.kernel(
        body,
        out_type=jax.ShapeDtypeStruct((NC * npad,), jnp.float32),
        mesh=mesh,
        compiler_params=pltpu.CompilerParams(needs_layout_passes=False),
        scratch_types=[
            pltpu.VMEM((ECHD,), jnp.int32), pltpu.VMEM((ECHD,), jnp.int32),
            pltpu.VMEM((ECHD,), jnp.int32), pltpu.VMEM((ECHD,), jnp.int32),
            pltpu.VMEM((ECHD,), jnp.float32), pltpu.VMEM((ECHD,), jnp.float32),
            pltpu.VMEM((ECHD,), jnp.float32), pltpu.VMEM((ECHD,), jnp.float32),
            pltpu.VMEM((span,), jnp.float32),
            pltpu.VMEM_SHARED((npad,), jnp.float32),
            pltpu.SemaphoreType.DMA, pltpu.SemaphoreType.DMA,
            pltpu.SemaphoreType.DMA, pltpu.SemaphoreType.DMA,
            pltpu.SemaphoreType.DMA, pltpu.SemaphoreType.DMA,
            pltpu.SemaphoreType.DMA, pltpu.SemaphoreType.DMA,
        ],
    )(ei1, w1)


@functools.partial(jax.jit, static_argnames=("npad", "span", "rows"))
def _run_sc_t(ei1, w1, u, *, npad, span, rows):
    mesh = plsc.VectorSubcoreMesh(core_axis_name="c", subcore_axis_name="s")
    body = functools.partial(_t_kernel_body, npad, span, rows)
    return pl.kernel(
        body,
        out_type=jax.ShapeDtypeStruct((NC * npad,), jnp.float32),
        mesh=mesh,
        compiler_params=pltpu.CompilerParams(needs_layout_passes=False),
        scratch_types=[
            pltpu.VMEM((ECHT,), jnp.int32), pltpu.VMEM((ECHT,), jnp.int32),
            pltpu.VMEM((ECHT,), jnp.int32), pltpu.VMEM((ECHT,), jnp.int32),
            pltpu.VMEM((ECHT,), jnp.int32), pltpu.VMEM((ECHT,), jnp.int32),
            pltpu.VMEM((ECHT,), jnp.int32), pltpu.VMEM((ECHT,), jnp.int32),
            pltpu.VMEM((ECHT,), jnp.float32), pltpu.VMEM((ECHT,), jnp.float32),
            pltpu.VMEM((ECHT,), jnp.float32), pltpu.VMEM((ECHT,), jnp.float32),
            pltpu.VMEM((ECHT,), jnp.float32), pltpu.VMEM((ECHT,), jnp.float32),
            pltpu.VMEM((ECHT,), jnp.float32), pltpu.VMEM((ECHT,), jnp.float32),
            pltpu.VMEM((span,), jnp.float32),
            pltpu.VMEM((npad,), jnp.float32),
            pltpu.VMEM_SHARED((npad,), jnp.float32),
            pltpu.SemaphoreType.DMA, pltpu.SemaphoreType.DMA,
            pltpu.SemaphoreType.DMA, pltpu.SemaphoreType.DMA,
            pltpu.SemaphoreType.DMA, pltpu.SemaphoreType.DMA,
            pltpu.SemaphoreType.DMA, pltpu.SemaphoreType.DMA,
        ],
    )(ei1, w1, u)


def kernel(x, edge_index, edge_weight, W_conv, b_conv,
           W1, b1, W2, b2, W3, b3, W4, b4, W5, b5):
    n = x.shape[0]
    e = edge_index.shape[1]
    assert e % LANES == 0
    rows = e // LANES
    span = -(-n // (NS * 32)) * 32
    npad = span * NS

    ei1 = edge_index.astype(jnp.int32).reshape(2 * rows * LANES)
    w1 = edge_weight.astype(jnp.float32)

    deg_p = _run_sc_deg(ei1, w1, npad=npad, span=span, rows=rows).reshape(NC, npad)
    deg = deg_p[0] + deg_p[1] + 1.0
    dinv = jnp.where(deg > 0, lax.rsqrt(jnp.where(deg > 0, deg, 1.0)), 0.0)
    xf = jnp.pad(x[:, 0].astype(jnp.float32), (0, npad - n))
    u = xf * dinv

    t_p = _run_sc_t(ei1, w1, u, npad=npad, span=span,
                    rows=rows).reshape(NC, npad)

    sb = dinv * (t_p[0] + t_p[1]) + dinv * dinv * xf
    zt = jnp.stack([sb[:n], xf[:n]])
    wc64 = jnp.concatenate([W_conv[0], jnp.zeros((1,), jnp.float32)]).reshape(64, 1)
    bc64 = jnp.concatenate([b_conv, jnp.zeros((1,), jnp.float32)]).reshape(64, 1)
    e64 = jnp.zeros((64, 1), jnp.float32).at[63, 0].set(1.0)

    bd = 12544
    grid = -(-n // bd)
    w64_spec = pl.BlockSpec((64, 64), lambda i: (0, 0))
    col_spec = pl.BlockSpec((64, 1), lambda i: (0, 0))

    out_t = pl.pallas_call(
        _mlp_kernel_body,
        grid=(grid,),
        in_specs=[pl.BlockSpec((2, bd), lambda i: (0, i)),
                  col_spec, col_spec, col_spec,
                  w64_spec, col_spec, w64_spec, col_spec, w64_spec, col_spec,
                  w64_spec, col_spec,
                  pl.BlockSpec((1, 64), lambda i: (0, 0)),
                  pl.BlockSpec((1, 1), lambda i: (0, 0))],
        out_specs=pl.BlockSpec((1, bd), lambda i: (0, i)),
        out_shape=jax.ShapeDtypeStruct((1, n), jnp.float32),
    )(zt, wc64, bc64, e64, W1.T, b1.reshape(64, 1), W2.T, b2.reshape(64, 1),
      W3.T, b3.reshape(64, 1), W4.T, b4.reshape(64, 1), W5.T, b5.reshape(1, 1))

    return out_t.reshape(n, 1)


# speedup vs baseline: 557.1755x; 1.0040x over previous
import functools

import jax
import jax.numpy as jnp
from jax import lax
from jax.experimental import pallas as pl
from jax.experimental.pallas import tpu as pltpu
from jax.experimental.pallas import tpu_sc as plsc

NC = 2
NS = 16
LANES = 128
RCD = 16
ECHD = RCD * LANES
RCT = 8
ECHT = RCT * LANES


def _row_split(rows, w, rc):
    nw = NC * NS
    blocks = rows // rc
    base = blocks // nw
    extra = blocks % nw
    start = rc * (w * base + jnp.minimum(w, extra))
    nrows = rc * (base + (w < extra).astype(jnp.int32))
    return start, nrows


def _deg_kernel_body(npad, span, rows, ei_hbm, w_hbm, out_hbm,
                     db0, db1, db2, db3, wb0, wb1, wb2, wb3, zbuf, deg_sh,
                     si0, si1, si2, si3, ss0, ss1, ss2, ss3):
    dbufs = (db0, db1, db2, db3)
    wbufs = (wb0, wb1, wb2, wb3)
    c = lax.axis_index("c")
    s = lax.axis_index("s")
    w = c * NS + s
    sin = (si0, si1, si2, si3)
    ssc = (ss0, ss1, ss2, ss3)

    def zb(i, _):
        zbuf[pl.ds(i * 16, 16)] = jnp.zeros((16,), jnp.float32)
        return 0
    lax.fori_loop(0, span // 16, zb, 0)
    pltpu.sync_copy(zbuf, deg_sh.at[pl.ds(s * span, span)])
    plsc.subcore_barrier()

    start, nrows = _row_split(rows, w, RCD)
    nchunk = nrows // RCD

    def start_in(k, b):
        e0 = (start + k * RCD) * LANES
        pltpu.async_copy(ei_hbm.at[pl.ds(rows * LANES + e0, ECHD)], dbufs[b],
                         sin[b])
        pltpu.async_copy(w_hbm.at[pl.ds(e0, ECHD)], wbufs[b], sin[b])

    def wait_in(b):
        pltpu.make_async_copy(ei_hbm.at[pl.ds(0, ECHD)], dbufs[b], sin[b]).wait()
        pltpu.make_async_copy(w_hbm.at[pl.ds(0, ECHD)], wbufs[b], sin[b]).wait()

    def fire_sc(b):
        pltpu.async_copy(wbufs[b], deg_sh.at[dbufs[b]], ssc[b], add=True)

    def drain_sc(b):
        pltpu.make_async_copy(wbufs[b], deg_sh.at[dbufs[b]], ssc[b]).wait()

    start_in(0, 0)
    start_in(1, 1)

    def body(k4, _):
        for b in range(4):
            k = k4 * 4 + b
            bn = (b + 2) % 4

            @pl.when((k >= 2) & (k - 2 < nchunk))
            def _():
                drain_sc(bn)

            @pl.when(k + 2 < nchunk)
            def _():
                start_in(k + 2, bn)

            @pl.when(k < nchunk)
            def _():
                wait_in(b)
                fire_sc(b)
        return 0
    lax.fori_loop(0, (nchunk + 5) // 4, body, 0)

    plsc.subcore_barrier()
    pltpu.sync_copy(deg_sh.at[pl.ds(s * span, span)],
                    out_hbm.at[pl.ds(c * npad + s * span, span)])


def _t_kernel_body(npad, span, rows, ei_hbm, w_hbm, u_hbm, out_hbm,
                   sb0, sb1, sb2, sb3, db0, db1, db2, db3,
                   wb0, wb1, wb2, wb3, pb0, pb1, pb2, pb3, zbuf, u_v, t_sh,
                   si0, si1, si2, si3, ss0, ss1, ss2, ss3):
    sbufs = (sb0, sb1, sb2, sb3)
    dbufs = (db0, db1, db2, db3)
    wbufs = (wb0, wb1, wb2, wb3)
    pbufs = (pb0, pb1, pb2, pb3)
    c = lax.axis_index("c")
    s = lax.axis_index("s")
    w = c * NS + s
    sin = (si0, si1, si2, si3)
    ssc = (ss0, ss1, ss2, ss3)

    def zb(i, _):
        zbuf[pl.ds(i * 16, 16)] = jnp.zeros((16,), jnp.float32)
        return 0
    lax.fori_loop(0, span // 16, zb, 0)
    pltpu.sync_copy(zbuf, t_sh.at[pl.ds(s * span, span)])
    pltpu.sync_copy(u_hbm, u_v)
    plsc.subcore_barrier()

    start, nrows = _row_split(rows, w, RCT)
    nchunk = nrows // RCT

    def start_in(k, b):
        e0 = (start + k * RCT) * LANES
        pltpu.async_copy(ei_hbm.at[pl.ds(e0, ECHT)], sbufs[b], sin[b])
        pltpu.async_copy(ei_hbm.at[pl.ds(rows * LANES + e0, ECHT)], dbufs[b],
                         sin[b])
        pltpu.async_copy(w_hbm.at[pl.ds(e0, ECHT)], wbufs[b], sin[b])

    def wait_in(b):
        pltpu.make_async_copy(ei_hbm.at[pl.ds(0, ECHT)], sbufs[b], sin[b]).wait()
        pltpu.make_async_copy(ei_hbm.at[pl.ds(0, ECHT)], dbufs[b], sin[b]).wait()
        pltpu.make_async_copy(w_hbm.at[pl.ds(0, ECHT)], wbufs[b], sin[b]).wait()

    def compute(b):
        for g in range(ECHT // 16):
            sl = pl.ds(g * 16, 16)
            gv = plsc.load_gather(u_v, [sbufs[b][sl]])
            pbufs[b][sl] = gv * wbufs[b][sl]

    def fire_sc(b):
        pltpu.async_copy(pbufs[b], t_sh.at[dbufs[b]], ssc[b], add=True)

    def drain_sc(b):
        pltpu.make_async_copy(pbufs[b], t_sh.at[dbufs[b]], ssc[b]).wait()

    start_in(0, 0)
    start_in(1, 1)

    def body(k4, _):
        for b in range(4):
            k = k4 * 4 + b
            bn = (b + 2) % 4

            @pl.when((k >= 2) & (k - 2 < nchunk))
            def _():
                drain_sc(bn)

            @pl.when(k + 2 < nchunk)
            def _():
                start_in(k + 2, bn)

            @pl.when(k < nchunk)
            def _():
                wait_in(b)
                compute(b)
                fire_sc(b)
        return 0
    lax.fori_loop(0, (nchunk + 5) // 4, body, 0)

    plsc.subcore_barrier()
    pltpu.sync_copy(t_sh.at[pl.ds(s * span, span)],
                    out_hbm.at[pl.ds(c * npad + s * span, span)])


def _mlp_kernel_body(zr, wc64r, bc64r, e64r,
                     w1r, b1r, w2r, b2r, w3r, b3r, w4r, b4r, w5r, b5r, outr):
    sb = zr[0:1, :]
    xb = zr[1:2, :]
    z = wc64r[...] * sb + e64r[...] * xb + bc64r[...]
    h = jnp.maximum(jnp.dot(w1r[...], z) + b1r[...], 0.0)
    h = jnp.maximum(jnp.dot(w2r[...], h) + b2r[...], 0.0)
    h = jnp.maximum(jnp.dot(w3r[...], h) + b3r[...], 0.0)
    h = jnp.maximum(jnp.dot(w4r[...], h) + b4r[...], 0.0)
    outr[...] = jnp.dot(w5r[...], h) + b5r[...]


@functools.partial(jax.jit, static_argnames=("npad", "span", "rows"))
def _run_sc_deg(ei1, w1, *, npad, span, rows):
    mesh = plsc.VectorSubcoreMesh(core_axis_name="c", subcore_axis_name="s")
    body = functools.partial(_deg_kernel_body, npad, span, rows)
    return pl.kernel(
        body,
        out_type=jax.ShapeDtypeStruct((NC * npad,), jnp.float32),
        mesh=mesh,
        compiler_params=pltpu.CompilerParams(needs_layout_passes=False),
        scratch_types=[
            pltpu.VMEM((ECHD,), jnp.int32), pltpu.VMEM((ECHD,), jnp.int32),
            pltpu.VMEM((ECHD,), jnp.int32), pltpu.VMEM((ECHD,), jnp.int32),
            pltpu.VMEM((ECHD,), jnp.float32), pltpu.VMEM((ECHD,), jnp.float32),
            pltpu.VMEM((ECHD,), jnp.float32), pltpu.VMEM((ECHD,), jnp.float32),
            pltpu.VMEM((span,), jnp.float32),
            pltpu.VMEM_SHARED((npad,), jnp.float32),
            pltpu.SemaphoreType.DMA, pltpu.SemaphoreType.DMA,
            pltpu.SemaphoreType.DMA, pltpu.SemaphoreType.DMA,
            pltpu.SemaphoreType.DMA, pltpu.SemaphoreType.DMA,
            pltpu.SemaphoreType.DMA, pltpu.SemaphoreType.DMA,
        ],
    )(ei1, w1)


@functools.partial(jax.jit, static_argnames=("npad", "span", "rows"))
def _run_sc_t(ei1, w1, u, *, npad, span, rows):
    mesh = plsc.VectorSubcoreMesh(core_axis_name="c", subcore_axis_name="s")
    body = functools.partial(_t_kernel_body, npad, span, rows)
    return pl.kernel(
        body,
        out_type=jax.ShapeDtypeStruct((NC * npad,), jnp.float32),
        mesh=mesh,
        compiler_params=pltpu.CompilerParams(needs_layout_passes=False),
        scratch_types=[
            pltpu.VMEM((ECHT,), jnp.int32), pltpu.VMEM((ECHT,), jnp.int32),
            pltpu.VMEM((ECHT,), jnp.int32), pltpu.VMEM((ECHT,), jnp.int32),
            pltpu.VMEM((ECHT,), jnp.int32), pltpu.VMEM((ECHT,), jnp.int32),
            pltpu.VMEM((ECHT,), jnp.int32), pltpu.VMEM((ECHT,), jnp.int32),
            pltpu.VMEM((ECHT,), jnp.float32), pltpu.VMEM((ECHT,), jnp.float32),
            pltpu.VMEM((ECHT,), jnp.float32), pltpu.VMEM((ECHT,), jnp.float32),
            pltpu.VMEM((ECHT,), jnp.float32), pltpu.VMEM((ECHT,), jnp.float32),
            pltpu.VMEM((ECHT,), jnp.float32), pltpu.VMEM((ECHT,), jnp.float32),
            pltpu.VMEM((span,), jnp.float32),
            pltpu.VMEM((npad,), jnp.float32),
            pltpu.VMEM_SHARED((npad,), jnp.float32),
            pltpu.SemaphoreType.DMA, pltpu.SemaphoreType.DMA,
            pltpu.SemaphoreType.DMA, pltpu.SemaphoreType.DMA,
            pltpu.SemaphoreType.DMA, pltpu.SemaphoreType.DMA,
            pltpu.SemaphoreType.DMA, pltpu.SemaphoreType.DMA,
        ],
    )(ei1, w1, u)


def kernel(x, edge_index, edge_weight, W_conv, b_conv,
           W1, b1, W2, b2, W3, b3, W4, b4, W5, b5):
    n = x.shape[0]
    e = edge_index.shape[1]
    assert e % LANES == 0
    rows = e // LANES
    span = -(-n // (NS * 32)) * 32
    npad = span * NS

    ei1 = edge_index.astype(jnp.int32).reshape(2 * rows * LANES)
    w1 = edge_weight.astype(jnp.float32)

    deg_p = _run_sc_deg(ei1, w1, npad=npad, span=span, rows=rows).reshape(NC, npad)
    deg = deg_p[0] + deg_p[1] + 1.0
    dinv = jnp.where(deg > 0, lax.rsqrt(jnp.where(deg > 0, deg, 1.0)), 0.0)
    xf = jnp.pad(x[:, 0].astype(jnp.float32), (0, npad - n))
    u = xf * dinv

    t_p = _run_sc_t(ei1, w1, u, npad=npad, span=span,
                    rows=rows).reshape(NC, npad)

    sb = dinv * (t_p[0] + t_p[1]) + dinv * dinv * xf
    zt = jnp.stack([sb[:n], xf[:n]])
    wc64 = jnp.concatenate([W_conv[0], jnp.zeros((1,), jnp.float32)]).reshape(64, 1)
    bc64 = jnp.concatenate([b_conv, jnp.zeros((1,), jnp.float32)]).reshape(64, 1)
    e64 = jnp.zeros((64, 1), jnp.float32).at[63, 0].set(1.0)

    bd = 25088
    grid = -(-n // bd)
    w64_spec = pl.BlockSpec((64, 64), lambda i: (0, 0))
    col_spec = pl.BlockSpec((64, 1), lambda i: (0, 0))

    out_t = pl.pallas_call(
        _mlp_kernel_body,
        grid=(grid,),
        in_specs=[pl.BlockSpec((2, bd), lambda i: (0, i)),
                  col_spec, col_spec, col_spec,
                  w64_spec, col_spec, w64_spec, col_spec, w64_spec, col_spec,
                  w64_spec, col_spec,
                  pl.BlockSpec((1, 64), lambda i: (0, 0)),
                  pl.BlockSpec((1, 1), lambda i: (0, 0))],
        out_specs=pl.BlockSpec((1, bd), lambda i: (0, i)),
        out_shape=jax.ShapeDtypeStruct((1, n), jnp.float32),
    )(zt, wc64, bc64, e64, W1.T, b1.reshape(64, 1), W2.T, b2.reshape(64, 1),
      W3.T, b3.reshape(64, 1), W4.T, b4.reshape(64, 1), W5.T, b5.reshape(1, 1))

    return out_t.reshape(n, 1)
